# baseline (speedup 1.0000x reference)
import functools

import jax
import jax.numpy as jnp
from jax import lax
from jax.experimental import pallas as pl
from jax.experimental.pallas import tpu as pltpu
from jax.experimental.pallas import tpu_sc as plsc

N_PTS = 10000
H = 32
C = 128
OUT = 128
KP = 15
KPP = 16
SIGMA = 0.1
E = N_PTS * H


_NC = 2
_NS = 16
_NW = _NC * _NS
_EPW = E // _NW
_CHUNK = 80
_NCHUNK = _EPW // _CHUNK


def _sc_gather_body(coords_hbm, feats_hbm, idx_hbm, spt_out, nf_out,
                    idx_v, sp_v, spt_v, nf_v, sem1, sem2):
    wid = lax.axis_index("s") * _NC + lax.axis_index("c")
    wbase = wid * _EPW
    lane = lax.iota(jnp.int32, 16)

    def chunk(j, carry):
        base = wbase + j * _CHUNK
        pltpu.sync_copy(idx_hbm.at[pl.ds(base, _CHUNK)], idx_v)
        cp1 = pltpu.async_copy(coords_hbm.at[idx_v], sp_v, sem1)
        cp2 = pltpu.async_copy(feats_hbm.at[idx_v], nf_v, sem2)
        cp1.wait()
        for c in range(3):
            ccol = jnp.full((16,), c, jnp.int32)
            for g in range(_CHUNK // 16):
                rows = lane + (g * 16)
                v = plsc.load_gather(sp_v, [rows, ccol])
                spt_v[c, pl.ds(g * 16, 16)] = v
        cp2.wait()
        pltpu.sync_copy(spt_v, spt_out.at[:, pl.ds(base, _CHUNK)])
        pltpu.sync_copy(nf_v, nf_out.at[pl.ds(base, _CHUNK)])
        return carry

    lax.fori_loop(0, _NCHUNK, chunk, 0)


@functools.lru_cache(maxsize=1)
def _make_sc_gather():
    return functools.partial(
        pl.kernel,
        mesh=plsc.VectorSubcoreMesh(core_axis_name="c", subcore_axis_name="s"),
        out_type=[
            jax.ShapeDtypeStruct((8, E), jnp.float32),
            jax.ShapeDtypeStruct((E, C), jnp.bfloat16),
        ],
        scratch_types=[
            pltpu.VMEM((_CHUNK,), jnp.int32),
            pltpu.VMEM((_CHUNK, 16), jnp.float32),
            pltpu.VMEM((8, _CHUNK), jnp.float32),
            pltpu.VMEM((_CHUNK, C), jnp.bfloat16),
            pltpu.SemaphoreType.DMA,
            pltpu.SemaphoreType.DMA,
        ],
        compiler_params=pltpu.CompilerParams(
            use_tc_tiling_on_sc=False, needs_layout_passes=False),
    )(_sc_gather_body)


_NB = 400
_EB = _NB * H
_SUB = 8
_ESUB = _SUB * H
_NSUB = _NB // _SUB


def _tc_body(spt_ref, qet_ref, nfg_ref, kpc_ref, w_ref, out_ref,
             acc_ref, p_ref):
    rq = lax.broadcasted_iota(jnp.int32, (_SUB * KPP, _ESUB), 0) // KPP
    cq = lax.broadcasted_iota(jnp.int32, (_SUB * KPP, _ESUB), 1) // H
    blockmask = rq == cq
    p_ref[...] = spt_ref[...] - qet_ref[...]
    kcb = [jnp.broadcast_to(kpc_ref[:, c:c + 1], (KPP, _ESUB))
           for c in range(3)]

    def sub(j2, carry):
        e00 = pl.multiple_of(j2 * 2 * _ESUB, 2 * _ESUB)
        halves = []
        for t in range(2):
            e0 = e00 + t * _ESUB
            sq = jnp.zeros((KPP, _ESUB), jnp.float32)
            for c in range(3):
                row = p_ref[c, pl.ds(e0, _ESUB)].reshape(1, _ESUB)
                d = jnp.broadcast_to(row, (KPP, _ESUB)) - kcb[c]
                sq = sq + d * d
            inflT = jnp.maximum(1.0 - jnp.sqrt(sq) / SIGMA, 0.0)
            a = jnp.broadcast_to(inflT[None], (_SUB, KPP, _ESUB))
            a = a.reshape(_SUB * KPP, _ESUB)
            a = jnp.where(blockmask, a, 0.0).astype(jnp.bfloat16)
            nf8 = nfg_ref[pl.ds(e0, _ESUB), :]
            w8 = jnp.dot(a, nf8, preferred_element_type=jnp.float32)
            halves.append(w8.reshape(_SUB, KPP * C))
        r0 = pl.multiple_of(j2 * 2 * _SUB, 2 * _SUB)
        acc_ref[pl.ds(r0, 2 * _SUB), :] = (
            jnp.concatenate(halves, axis=0).astype(jnp.bfloat16))
        return carry

    lax.fori_loop(0, _NSUB // 2, sub, 0)
    out = jnp.dot(acc_ref[...], w_ref[...], preferred_element_type=jnp.float32)
    out_ref[...] = jnp.where(out >= 0, out, 0.1 * out)


def _tc_compute(spt, qet, nf_g, kpc, wflat):
    return pl.pallas_call(
        _tc_body,
        grid=(N_PTS // _NB,),
        in_specs=[
            pl.BlockSpec((8, _EB), lambda i: (0, i)),
            pl.BlockSpec((8, _EB), lambda i: (0, i)),
            pl.BlockSpec((_EB, C), lambda i: (i, 0)),
            pl.BlockSpec((KPP, 16), lambda i: (0, 0)),
            pl.BlockSpec((KPP * C, OUT), lambda i: (0, 0)),
        ],
        out_specs=pl.BlockSpec((_NB, OUT), lambda i: (i, 0)),
        out_shape=jax.ShapeDtypeStruct((N_PTS, OUT), jnp.float32),
        scratch_shapes=[
            pltpu.VMEM((_NB, KPP * C), jnp.bfloat16),
            pltpu.VMEM((8, _EB), jnp.float32),
        ],
    )(spt, qet, nf_g, kpc, wflat)


def kernel(q_points, s_points, feats, neighbor_indices, kernel_points, weights):
    ni = neighbor_indices.reshape(-1).astype(jnp.int32)
    coords = jnp.pad(s_points, ((0, 0), (0, 13)))
    qet = jnp.zeros((8, E), jnp.float32)
    qet = qet.at[0:3, :].set(jnp.repeat(q_points.T, H, axis=1))
    kpp = jnp.concatenate(
        [kernel_points, jnp.full((1, 3), 1e3, jnp.float32)], axis=0)
    kpc = jnp.pad(kpp, ((0, 0), (0, 13)))
    wflat = jnp.concatenate(
        [weights, jnp.zeros((1, C, OUT), weights.dtype)], axis=0
    ).reshape(KPP * C, OUT).astype(jnp.bfloat16)

    spt, nf_g = _make_sc_gather()(coords, feats.astype(jnp.bfloat16), ni)
    return _tc_compute(spt, qet, nf_g, kpc, wflat)


# speedup vs baseline: 2.2104x; 2.2104x over previous
import functools

import jax
import jax.numpy as jnp
from jax import lax
from jax.experimental import pallas as pl
from jax.experimental.pallas import tpu as pltpu
from jax.experimental.pallas import tpu_sc as plsc

N_PTS = 10000
H = 32
C = 128
OUT = 128
KP = 15
KPP = 16
SIGMA = 0.1
E = N_PTS * H


_NC = 2
_NS = 16
_NW = _NC * _NS
_EPW = E // _NW
_CHUNK = 80
_NCHUNK = _EPW // _CHUNK


def _sc_gather_body(coords_hbm, feats_hbm, idx_hbm, spt_out, nf_out,
                    idx_all, sp0, sp1, spt0, spt1, nf0, nf1,
                    semc0, semc1, semf0, semf1,
                    semos0, semos1, semon0, semon1):
    wid = lax.axis_index("s") * _NC + lax.axis_index("c")
    wbase = wid * _EPW
    lane = lax.iota(jnp.int32, 16)
    sp = [sp0, sp1]
    spt = [spt0, spt1]
    nf = [nf0, nf1]
    semc = [semc0, semc1]
    semf = [semf0, semf1]
    semos = [semos0, semos1]
    semon = [semon0, semon1]

    pltpu.sync_copy(idx_hbm.at[pl.ds(wbase, _EPW)], idx_all)

    def idxs(j):
        return idx_all.at[pl.ds(j * _CHUNK, _CHUNK)]

    def fire(j, b):
        pltpu.async_copy(coords_hbm.at[idxs(j)], sp[b], semc[b])
        pltpu.async_copy(feats_hbm.at[idxs(j)], nf[b], semf[b])

    def wait_gather(j, b):
        pltpu.make_async_copy(coords_hbm.at[idxs(j)], sp[b], semc[b]).wait()
        pltpu.make_async_copy(feats_hbm.at[idxs(j)], nf[b], semf[b]).wait()

    def process(j, b):
        base = wbase + j * _CHUNK
        for c in range(3):
            ccol = jnp.full((16,), c, jnp.int32)
            for g in range(_CHUNK // 16):
                v = plsc.load_gather(sp[b], [lane + g * 16, ccol])
                spt[b][c, pl.ds(g * 16, 16)] = v
        pltpu.async_copy(spt[b], spt_out.at[:, pl.ds(base, _CHUNK)], semos[b])
        pltpu.async_copy(nf[b], nf_out.at[pl.ds(base, _CHUNK)], semon[b])

    def wait_out(j, b):
        base = wbase + j * _CHUNK
        pltpu.make_async_copy(
            spt[b], spt_out.at[:, pl.ds(base, _CHUNK)], semos[b]).wait()
        pltpu.make_async_copy(
            nf[b], nf_out.at[pl.ds(base, _CHUNK)], semon[b]).wait()

    fire(0, 0)

    def outer(j2, carry):
        for t in range(2):
            j = j2 * 2 + t
            b = t
            bn = 1 - t

            @pl.when(j >= 1)
            def _():
                wait_out(j - 1, bn)

            fire(j + 1, bn)
            wait_gather(j, b)
            process(j, b)
        return carry

    lax.fori_loop(0, (_NCHUNK - 1) // 2, outer, 0)
    wait_out(_NCHUNK - 2, 1)
    wait_gather(_NCHUNK - 1, 0)
    process(_NCHUNK - 1, 0)
    wait_out(_NCHUNK - 1, 0)


@functools.lru_cache(maxsize=1)
def _make_sc_gather():
    return functools.partial(
        pl.kernel,
        mesh=plsc.VectorSubcoreMesh(core_axis_name="c", subcore_axis_name="s"),
        out_type=[
            jax.ShapeDtypeStruct((8, E), jnp.float32),
            jax.ShapeDtypeStruct((E, C), jnp.float32),
        ],
        scratch_types=[
            pltpu.VMEM((_EPW,), jnp.int32),
            pltpu.VMEM((_CHUNK, 16), jnp.float32),
            pltpu.VMEM((_CHUNK, 16), jnp.float32),
            pltpu.VMEM((8, _CHUNK), jnp.float32),
            pltpu.VMEM((8, _CHUNK), jnp.float32),
            pltpu.VMEM((_CHUNK, C), jnp.float32),
            pltpu.VMEM((_CHUNK, C), jnp.float32),
            pltpu.SemaphoreType.DMA,
            pltpu.SemaphoreType.DMA,
            pltpu.SemaphoreType.DMA,
            pltpu.SemaphoreType.DMA,
            pltpu.SemaphoreType.DMA,
            pltpu.SemaphoreType.DMA,
            pltpu.SemaphoreType.DMA,
            pltpu.SemaphoreType.DMA,
        ],
        compiler_params=pltpu.CompilerParams(
            use_tc_tiling_on_sc=False, needs_layout_passes=False),
    )(_sc_gather_body)


_NB = 400
_EB = _NB * H
_SUB = 8
_ESUB = _SUB * H
_NSUB = _NB // _SUB


def _tc_body(spt_ref, qet_ref, nfg_ref, kpc_ref, w_ref, out_ref,
             acc_ref, p_ref):
    rq = lax.broadcasted_iota(jnp.int32, (_SUB * KPP, _ESUB), 0) // KPP
    cq = lax.broadcasted_iota(jnp.int32, (_SUB * KPP, _ESUB), 1) // H
    blockmask = rq == cq
    p_ref[...] = spt_ref[...] - qet_ref[...]
    kcb = [jnp.broadcast_to(kpc_ref[:, c:c + 1], (KPP, _ESUB))
           for c in range(3)]

    def sub(j2, carry):
        e00 = pl.multiple_of(j2 * 2 * _ESUB, 2 * _ESUB)
        halves = []
        for t in range(2):
            e0 = e00 + t * _ESUB
            sq = jnp.zeros((KPP, _ESUB), jnp.float32)
            for c in range(3):
                row = p_ref[c, pl.ds(e0, _ESUB)].reshape(1, _ESUB)
                d = jnp.broadcast_to(row, (KPP, _ESUB)) - kcb[c]
                sq = sq + d * d
            inflT = jnp.maximum(1.0 - jnp.sqrt(sq) / SIGMA, 0.0)
            a = jnp.broadcast_to(inflT[None], (_SUB, KPP, _ESUB))
            a = a.reshape(_SUB * KPP, _ESUB)
            a = jnp.where(blockmask, a, 0.0).astype(jnp.bfloat16)
            nf8 = nfg_ref[pl.ds(e0, _ESUB), :].astype(jnp.bfloat16)
            w8 = jnp.dot(a, nf8, preferred_element_type=jnp.float32)
            halves.append(w8.reshape(_SUB, KPP * C))
        r0 = pl.multiple_of(j2 * 2 * _SUB, 2 * _SUB)
        acc_ref[pl.ds(r0, 2 * _SUB), :] = (
            jnp.concatenate(halves, axis=0).astype(jnp.bfloat16))
        return carry

    lax.fori_loop(0, _NSUB // 2, sub, 0)
    out = jnp.dot(acc_ref[...], w_ref[...], preferred_element_type=jnp.float32)
    out_ref[...] = jnp.where(out >= 0, out, 0.1 * out)


def _tc_compute(spt, qet, nf_g, kpc, wflat):
    return pl.pallas_call(
        _tc_body,
        grid=(N_PTS // _NB,),
        in_specs=[
            pl.BlockSpec((8, _EB), lambda i: (0, i)),
            pl.BlockSpec((8, _EB), lambda i: (0, i)),
            pl.BlockSpec((_EB, C), lambda i: (i, 0)),
            pl.BlockSpec((KPP, 16), lambda i: (0, 0)),
            pl.BlockSpec((KPP * C, OUT), lambda i: (0, 0)),
        ],
        out_specs=pl.BlockSpec((_NB, OUT), lambda i: (i, 0)),
        out_shape=jax.ShapeDtypeStruct((N_PTS, OUT), jnp.float32),
        scratch_shapes=[
            pltpu.VMEM((_NB, KPP * C), jnp.bfloat16),
            pltpu.VMEM((8, _EB), jnp.float32),
        ],
    )(spt, qet, nf_g, kpc, wflat)


def kernel(q_points, s_points, feats, neighbor_indices, kernel_points, weights):
    ni = neighbor_indices.reshape(-1).astype(jnp.int32)
    coords = jnp.pad(s_points, ((0, 0), (0, 13)))
    qet = jnp.zeros((8, E), jnp.float32)
    qet = qet.at[0:3, :].set(jnp.repeat(q_points.T, H, axis=1))
    kpp = jnp.concatenate(
        [kernel_points, jnp.full((1, 3), 1e3, jnp.float32)], axis=0)
    kpc = jnp.pad(kpp, ((0, 0), (0, 13)))
    wflat = jnp.concatenate(
        [weights, jnp.zeros((1, C, OUT), weights.dtype)], axis=0
    ).reshape(KPP * C, OUT).astype(jnp.bfloat16)

    spt, nf_g = _make_sc_gather()(coords, feats, ni)
    return _tc_compute(spt, qet, nf_g, kpc, wflat)
